# baseline (speedup 1.0000x reference)
import jax
import jax.numpy as jnp
from jax.experimental import pallas as pl
from jax.experimental.pallas import tpu as pltpu

POOL = 64
PLEN = 8
TOPK = 8
HEADS = 4
BLOCK_B = 512

_NT = (((1,), (1,)), ((), ()))


def _main_kernel(x_ref, keys_ref, vals_ref, w_in_ref, b_in_ref, wo_ref,
                 bo_ref, lnw_ref, lnb_ref, out_ref,
                 kn_s, vk_s, vv_s, wq_s, wo_s):
    D = x_ref.shape[1]
    dh = D // HEADS

    @pl.when(pl.program_id(0) == 0)
    def _prep():
        k = keys_ref[...]
        n = jnp.sqrt(jnp.sum(k * k, axis=1, keepdims=True))
        kn_s[...] = k / jnp.maximum(n, 1e-12)
        v = vals_ref[...].astype(jnp.bfloat16)
        wk = w_in_ref[D:2 * D, :].astype(jnp.bfloat16)
        wv = w_in_ref[2 * D:, :].astype(jnp.bfloat16)
        vk = jax.lax.dot_general(v, wk, _NT, preferred_element_type=jnp.float32)
        vv = jax.lax.dot_general(v, wv, _NT, preferred_element_type=jnp.float32)
        vk_s[...] = (vk + b_in_ref[:, D:2 * D]).astype(jnp.bfloat16)
        vv_s[...] = (vv + b_in_ref[:, 2 * D:]).astype(jnp.bfloat16)
        wq_s[...] = w_in_ref[:D, :].astype(jnp.bfloat16)
        wo_s[...] = wo_ref[...].astype(jnp.bfloat16)

    x = x_ref[...]
    simT = jax.lax.dot_general(kn_s[...], x, _NT,
                               preferred_element_type=jnp.float32)

    iota_p = jax.lax.broadcasted_iota(jnp.int32, simT.shape, 0)
    cur = simT
    sel = jnp.zeros_like(simT)
    for _ in range(TOPK):
        m = jnp.max(cur, axis=0, keepdims=True)
        idx = jnp.where(cur == m, iota_p, POOL)
        jmin = jnp.min(idx, axis=0, keepdims=True)
        hit = iota_p == jmin
        sel = jnp.where(hit, 1.0, sel)
        cur = jnp.where(hit, -jnp.inf, cur)
    mask_pool = sel.T
    mask_full = jnp.concatenate([mask_pool] * PLEN, axis=1) > 0.5

    qh = jax.lax.dot_general(x.astype(jnp.bfloat16), wq_s[...], _NT,
                             preferred_element_type=jnp.float32) + b_in_ref[:, :D]
    scale = jnp.float32(1.4426950408889634) / jnp.sqrt(jnp.float32(dh))
    qh = (qh * scale).astype(jnp.bfloat16)
    vk = vk_s[...]
    vv = vv_s[...]
    ctxs = []
    for h in range(HEADS):
        sl = slice(h * dh, (h + 1) * dh)
        lg = jax.lax.dot_general(qh[:, sl], vk[:, sl], _NT,
                                 preferred_element_type=jnp.float32)
        lg = jnp.where(mask_full, lg, -1e30)
        m = jnp.max(lg, axis=1, keepdims=True)
        e = jax.lax.exp2(lg - m)
        inv = 1.0 / jnp.sum(e, axis=1, keepdims=True)
        cu = jnp.dot(e.astype(jnp.bfloat16), vv[:, sl],
                     preferred_element_type=jnp.float32)
        ctxs.append(cu * inv)
    ctx = jnp.concatenate(ctxs, axis=1).astype(jnp.bfloat16)

    attended = jax.lax.dot_general(ctx, wo_s[...], _NT,
                                   preferred_element_type=jnp.float32) + bo_ref[...]
    y = x + attended
    mu = jnp.mean(y, axis=1, keepdims=True)
    var = jnp.mean(y * y, axis=1, keepdims=True) - mu * mu
    out_ref[...] = (y - mu) / jnp.sqrt(var + 1e-5) * lnw_ref[...] + lnb_ref[...]


def kernel(x, keys, values, in_proj_weight, in_proj_bias, out_proj_weight,
           out_proj_bias, ln_weight, ln_bias):
    Bc, D = x.shape
    R = POOL * PLEN
    b_in = in_proj_bias.reshape(1, 3 * D)
    bo = out_proj_bias.reshape(1, D)
    lnw = ln_weight.reshape(1, D)
    lnb = ln_bias.reshape(1, D)
    vals2d = values.transpose(1, 0, 2).reshape(R, D)

    nb = Bc // BLOCK_B
    full = lambda i: (0, 0)
    out = pl.pallas_call(
        _main_kernel,
        grid=(nb,),
        in_specs=[
            pl.BlockSpec((BLOCK_B, D), lambda i: (i, 0)),
            pl.BlockSpec((POOL, D), full),
            pl.BlockSpec((R, D), full),
            pl.BlockSpec((3 * D, D), full),
            pl.BlockSpec((1, 3 * D), full),
            pl.BlockSpec((D, D), full),
            pl.BlockSpec((1, D), full),
            pl.BlockSpec((1, D), full),
            pl.BlockSpec((1, D), full),
        ],
        out_specs=pl.BlockSpec((BLOCK_B, D), lambda i: (i, 0)),
        out_shape=jax.ShapeDtypeStruct((Bc, D), jnp.float32),
        scratch_shapes=[
            pltpu.VMEM((POOL, D), jnp.float32),
            pltpu.VMEM((R, D), jnp.bfloat16),
            pltpu.VMEM((R, D), jnp.bfloat16),
            pltpu.VMEM((D, D), jnp.bfloat16),
            pltpu.VMEM((D, D), jnp.bfloat16),
        ],
    )(x, keys, vals2d, in_proj_weight, b_in, out_proj_weight, bo, lnw, lnb)
    return out


# speedup vs baseline: 1.0128x; 1.0128x over previous
import jax
import jax.numpy as jnp
from jax.experimental import pallas as pl
from jax.experimental.pallas import tpu as pltpu

POOL = 64
PLEN = 8
TOPK = 8
HEADS = 4
BLOCK_B = 1024

_NT = (((1,), (1,)), ((), ()))


def _main_kernel(x_ref, keys_ref, vals_ref, w_in_ref, b_in_ref, wo_ref,
                 bo_ref, lnw_ref, lnb_ref, out_ref,
                 kn_s, vk_s, vv_s, wq_s, wo_s):
    D = x_ref.shape[1]
    dh = D // HEADS

    @pl.when(pl.program_id(0) == 0)
    def _prep():
        k = keys_ref[...]
        n = jnp.sqrt(jnp.sum(k * k, axis=1, keepdims=True))
        kn_s[...] = k / jnp.maximum(n, 1e-12)
        v = vals_ref[...].astype(jnp.bfloat16)
        wk = w_in_ref[D:2 * D, :].astype(jnp.bfloat16)
        wv = w_in_ref[2 * D:, :].astype(jnp.bfloat16)
        vk = jax.lax.dot_general(v, wk, _NT, preferred_element_type=jnp.float32)
        vv = jax.lax.dot_general(v, wv, _NT, preferred_element_type=jnp.float32)
        vk_s[...] = (vk + b_in_ref[:, D:2 * D]).astype(jnp.bfloat16)
        vv_s[...] = (vv + b_in_ref[:, 2 * D:]).astype(jnp.bfloat16)
        wq_s[...] = w_in_ref[:D, :].astype(jnp.bfloat16)
        wo_s[...] = wo_ref[...].astype(jnp.bfloat16)

    x = x_ref[...]
    simT = jax.lax.dot_general(kn_s[...], x, _NT,
                               preferred_element_type=jnp.float32)

    iota_p = jax.lax.broadcasted_iota(jnp.int32, simT.shape, 0)
    cur = simT
    sel = jnp.zeros_like(simT)
    for _ in range(TOPK):
        m = jnp.max(cur, axis=0, keepdims=True)
        idx = jnp.where(cur == m, iota_p, POOL)
        jmin = jnp.min(idx, axis=0, keepdims=True)
        hit = iota_p == jmin
        sel = jnp.where(hit, 1.0, sel)
        cur = jnp.where(hit, -jnp.inf, cur)
    mask_pool = sel.T
    mask_full = jnp.concatenate([mask_pool] * PLEN, axis=1) > 0.5

    qh = jax.lax.dot_general(x.astype(jnp.bfloat16), wq_s[...], _NT,
                             preferred_element_type=jnp.float32) + b_in_ref[:, :D]
    scale = jnp.float32(1.4426950408889634) / jnp.sqrt(jnp.float32(dh))
    qh = (qh * scale).astype(jnp.bfloat16)
    vk = vk_s[...]
    vv = vv_s[...]
    ctxs = []
    for h in range(HEADS):
        sl = slice(h * dh, (h + 1) * dh)
        lg = jax.lax.dot_general(qh[:, sl], vk[:, sl], _NT,
                                 preferred_element_type=jnp.float32)
        lg = jnp.where(mask_full, lg, -1e30)
        m = jnp.max(lg, axis=1, keepdims=True)
        e = jax.lax.exp2(lg - m)
        inv = 1.0 / jnp.sum(e, axis=1, keepdims=True)
        cu = jnp.dot(e.astype(jnp.bfloat16), vv[:, sl],
                     preferred_element_type=jnp.float32)
        ctxs.append(cu * inv)
    ctx = jnp.concatenate(ctxs, axis=1).astype(jnp.bfloat16)

    attended = jax.lax.dot_general(ctx, wo_s[...], _NT,
                                   preferred_element_type=jnp.float32) + bo_ref[...]
    y = x + attended
    mu = jnp.mean(y, axis=1, keepdims=True)
    var = jnp.mean(y * y, axis=1, keepdims=True) - mu * mu
    out_ref[...] = (y - mu) / jnp.sqrt(var + 1e-5) * lnw_ref[...] + lnb_ref[...]


def kernel(x, keys, values, in_proj_weight, in_proj_bias, out_proj_weight,
           out_proj_bias, ln_weight, ln_bias):
    Bc, D = x.shape
    R = POOL * PLEN
    b_in = in_proj_bias.reshape(1, 3 * D)
    bo = out_proj_bias.reshape(1, D)
    lnw = ln_weight.reshape(1, D)
    lnb = ln_bias.reshape(1, D)
    vals2d = values.transpose(1, 0, 2).reshape(R, D)

    nb = Bc // BLOCK_B
    full = lambda i: (0, 0)
    out = pl.pallas_call(
        _main_kernel,
        grid=(nb,),
        in_specs=[
            pl.BlockSpec((BLOCK_B, D), lambda i: (i, 0)),
            pl.BlockSpec((POOL, D), full),
            pl.BlockSpec((R, D), full),
            pl.BlockSpec((3 * D, D), full),
            pl.BlockSpec((1, 3 * D), full),
            pl.BlockSpec((D, D), full),
            pl.BlockSpec((1, D), full),
            pl.BlockSpec((1, D), full),
            pl.BlockSpec((1, D), full),
        ],
        out_specs=pl.BlockSpec((BLOCK_B, D), lambda i: (i, 0)),
        out_shape=jax.ShapeDtypeStruct((Bc, D), jnp.float32),
        scratch_shapes=[
            pltpu.VMEM((POOL, D), jnp.float32),
            pltpu.VMEM((R, D), jnp.bfloat16),
            pltpu.VMEM((R, D), jnp.bfloat16),
            pltpu.VMEM((D, D), jnp.bfloat16),
            pltpu.VMEM((D, D), jnp.bfloat16),
        ],
    )(x, keys, vals2d, in_proj_weight, b_in, out_proj_weight, bo, lnw, lnb)
    return out
